# baseline (speedup 1.0000x reference)
import jax
import jax.numpy as jnp
from jax.experimental import pallas as pl
from jax.experimental.pallas import tpu as pltpu

N = 1024
D = 128
N_STEPS = 2
ETA = 0.1
SIGMA = 1.0

BNS = 64
KS = N // BNS


def _matvec_from_wt(wt_bf, mu_blk, b_blk, scale=1.0):
    t = wt_bf * mu_blk.astype(jnp.bfloat16)[:, :, None]
    while t.shape[1] > 8:
        h = t.shape[1] // 2
        t = t[:, :h, :] + t[:, h:, :]
    s = jnp.sum(t.astype(jnp.float32), axis=1)
    return jnp.tanh(scale * s + b_blk)


def _mega_kernel(adj_ref, obs_ref, maskf_ref, b_ref, W_hbm,
                 mu_ref, eps_ref, fe_ref,
                 A_ref, wagg_ref, rnc_ref, invm_ref, preds_ref,
                 Wb_ref, stage_ref, sem_ref):
    f32 = jnp.float32

    A = (adj_ref[...] > 0).astype(jnp.bfloat16)
    A_ref[...] = A
    ones = jnp.ones((N, 1), jnp.bfloat16)
    n_par = jax.lax.dot_general(A, ones, (((0,), (0,)), ((), ())),
                                preferred_element_type=f32)
    n_chi = jax.lax.dot_general(A, ones, (((1,), (0,)), ((), ())),
                                preferred_element_type=f32)
    wagg_ref[...] = jnp.where(n_par > 0, 1.0 / jnp.maximum(n_par, 1.0), 0.0)
    rnc_ref[...] = 0.5 / jnp.maximum(n_chi, 1.0)
    maskf = maskf_ref[...]
    invm_ref[...] = 1.0 - maskf
    mu_ref[...] = obs_ref[...] * maskf

    def w_copy(k, buf):
        return pltpu.make_async_copy(
            W_hbm.at[pl.ds(k * BNS, BNS)], stage_ref.at[buf], sem_ref.at[buf])

    w_copy(0, 0).start()

    def body1(k, carry):
        buf = jax.lax.rem(k, 2)
        nbuf = jax.lax.rem(k + 1, 2)

        @pl.when(k + 1 < KS)
        def _():
            w_copy(k + 1, nbuf).start()

        w_copy(k, buf).wait()
        wt_bf = jnp.swapaxes(stage_ref[buf].astype(jnp.bfloat16), 1, 2)
        sl = pl.ds(k * BNS, BNS)
        Wb_ref[sl] = (wt_bf * 16.0).astype(jnp.float8_e4m3fn)
        preds_ref[sl, :] = _matvec_from_wt(wt_bf, mu_ref[sl, :], b_ref[sl, :])
        return carry

    jax.lax.fori_loop(0, KS, body1, 0, unroll=2)

    def update(last):
        A = A_ref[...]
        agg = jax.lax.dot_general(A, preds_ref[...].astype(jnp.bfloat16),
                                  (((0,), (0,)), ((), ())),
                                  preferred_element_type=f32)
        mu = mu_ref[...]
        eps = mu - wagg_ref[...] * agg
        child = jax.lax.dot_general(A, eps.astype(jnp.bfloat16),
                                    (((1,), (0,)), ((), ())),
                                    preferred_element_type=f32)
        upd = -eps + child * rnc_ref[...]
        mu_ref[...] = invm_ref[...] * (mu + ETA * upd) + obs_ref[...] * maskf_ref[...]
        eps_ref[...] = eps
        if last:
            fe_ref[...] = (0.5 / (SIGMA * SIGMA)) * jnp.sum(eps * eps).reshape(1, 1)

    update(False)

    BN2 = 128
    def body2(k, carry):
        sl = pl.ds(k * BN2, BN2)
        preds_ref[sl, :] = _matvec_from_wt(Wb_ref[sl].astype(jnp.bfloat16),
                                           mu_ref[sl, :], b_ref[sl, :],
                                           scale=1.0 / 16.0)
        return carry

    jax.lax.fori_loop(0, N // BN2, body2, 0, unroll=2)

    update(True)


def kernel(observations, adj_matrix, obs_mask, W, b):
    obs = observations.astype(jnp.float32)
    maskf = obs_mask.astype(jnp.float32).reshape(N, 1)
    fdt = jax.ShapeDtypeStruct
    vmem = pltpu.MemorySpace.VMEM

    mu, eps, fe = pl.pallas_call(
        _mega_kernel,
        in_specs=[
            pl.BlockSpec(memory_space=vmem),
            pl.BlockSpec(memory_space=vmem),
            pl.BlockSpec(memory_space=vmem),
            pl.BlockSpec(memory_space=vmem),
            pl.BlockSpec(memory_space=pltpu.MemorySpace.HBM),
        ],
        out_specs=(pl.BlockSpec(memory_space=vmem),) * 3,
        out_shape=(fdt((N, D), jnp.float32), fdt((N, D), jnp.float32),
                   fdt((1, 1), jnp.float32)),
        scratch_shapes=[
            pltpu.VMEM((N, N), jnp.bfloat16),
            pltpu.VMEM((N, 1), jnp.float32),
            pltpu.VMEM((N, 1), jnp.float32),
            pltpu.VMEM((N, 1), jnp.float32),
            pltpu.VMEM((N, D), jnp.float32),
            pltpu.VMEM((N, D, D), jnp.float8_e4m3fn),
            pltpu.VMEM((2, BNS, D, D), jnp.float32),
            pltpu.SemaphoreType.DMA((2,)),
        ],
        compiler_params=pltpu.CompilerParams(
            vmem_limit_bytes=100 * 1024 * 1024),
    )(adj_matrix, obs, maskf, b, W)

    return (mu, eps, fe.reshape(()))


# speedup vs baseline: 1.0122x; 1.0122x over previous
import jax
import jax.numpy as jnp
from jax.experimental import pallas as pl
from jax.experimental.pallas import tpu as pltpu

N = 1024
D = 128
N_STEPS = 2
ETA = 0.1
SIGMA = 1.0

BNS = 64
KS = N // BNS


def _matvec_from_wt(wt_bf, mu_blk, b_blk):
    t = wt_bf * mu_blk.astype(jnp.bfloat16)[:, :, None]
    while t.shape[1] > 8:
        h = t.shape[1] // 2
        t = t[:, :h, :] + t[:, h:, :]
    s = jnp.sum(t.astype(jnp.float32), axis=1)
    return jnp.tanh(s + b_blk)


def _mega_kernel(adj_ref, obs_ref, maskf_ref, b_ref, W_hbm,
                 mu_ref, eps_ref, fe_ref,
                 A_ref, wagg_ref, rnc_ref, invm_ref, preds_ref,
                 Wb_ref, stage_ref, sem_ref):
    f32 = jnp.float32

    A = (adj_ref[...] > 0).astype(jnp.bfloat16)
    A_ref[...] = A
    ones = jnp.ones((N, 1), jnp.bfloat16)
    n_par = jax.lax.dot_general(A, ones, (((0,), (0,)), ((), ())),
                                preferred_element_type=f32)
    n_chi = jax.lax.dot_general(A, ones, (((1,), (0,)), ((), ())),
                                preferred_element_type=f32)
    wagg_ref[...] = jnp.where(n_par > 0, 1.0 / jnp.maximum(n_par, 1.0), 0.0)
    rnc_ref[...] = 0.5 / jnp.maximum(n_chi, 1.0)
    maskf = maskf_ref[...]
    invm_ref[...] = 1.0 - maskf
    mu_ref[...] = obs_ref[...] * maskf

    def w_copy(k, buf):
        return pltpu.make_async_copy(
            W_hbm.at[pl.ds(k * BNS, BNS)], stage_ref.at[buf], sem_ref.at[buf])

    w_copy(0, 0).start()

    def body1(k, carry):
        buf = jax.lax.rem(k, 2)
        nbuf = jax.lax.rem(k + 1, 2)

        @pl.when(k + 1 < KS)
        def _():
            w_copy(k + 1, nbuf).start()

        w_copy(k, buf).wait()
        wt_bf = jnp.swapaxes(stage_ref[buf].astype(jnp.bfloat16), 1, 2)
        sl = pl.ds(k * BNS, BNS)
        Wb_ref[sl] = wt_bf
        preds_ref[sl, :] = _matvec_from_wt(wt_bf, mu_ref[sl, :], b_ref[sl, :])
        return carry

    jax.lax.fori_loop(0, KS, body1, 0, unroll=2)

    def update(last):
        A = A_ref[...]
        agg = jax.lax.dot_general(A, preds_ref[...].astype(jnp.bfloat16),
                                  (((0,), (0,)), ((), ())),
                                  preferred_element_type=f32)
        mu = mu_ref[...]
        eps = mu - wagg_ref[...] * agg
        child = jax.lax.dot_general(A, eps.astype(jnp.bfloat16),
                                    (((1,), (0,)), ((), ())),
                                    preferred_element_type=f32)
        upd = -eps + child * rnc_ref[...]
        mu_ref[...] = invm_ref[...] * (mu + ETA * upd) + obs_ref[...] * maskf_ref[...]
        eps_ref[...] = eps
        if last:
            fe_ref[...] = (0.5 / (SIGMA * SIGMA)) * jnp.sum(eps * eps).reshape(1, 1)

    update(False)

    BN2 = 128
    def body2(k, carry):
        sl = pl.ds(k * BN2, BN2)
        preds_ref[sl, :] = _matvec_from_wt(Wb_ref[sl], mu_ref[sl, :], b_ref[sl, :])
        return carry

    jax.lax.fori_loop(0, N // BN2, body2, 0, unroll=8)

    update(True)


def kernel(observations, adj_matrix, obs_mask, W, b):
    obs = observations.astype(jnp.float32)
    maskf = obs_mask.astype(jnp.float32).reshape(N, 1)
    fdt = jax.ShapeDtypeStruct
    vmem = pltpu.MemorySpace.VMEM

    mu, eps, fe = pl.pallas_call(
        _mega_kernel,
        in_specs=[
            pl.BlockSpec(memory_space=vmem),
            pl.BlockSpec(memory_space=vmem),
            pl.BlockSpec(memory_space=vmem),
            pl.BlockSpec(memory_space=vmem),
            pl.BlockSpec(memory_space=pltpu.MemorySpace.HBM),
        ],
        out_specs=(pl.BlockSpec(memory_space=vmem),) * 3,
        out_shape=(fdt((N, D), jnp.float32), fdt((N, D), jnp.float32),
                   fdt((1, 1), jnp.float32)),
        scratch_shapes=[
            pltpu.VMEM((N, N), jnp.bfloat16),
            pltpu.VMEM((N, 1), jnp.float32),
            pltpu.VMEM((N, 1), jnp.float32),
            pltpu.VMEM((N, 1), jnp.float32),
            pltpu.VMEM((N, D), jnp.float32),
            pltpu.VMEM((N, D, D), jnp.bfloat16),
            pltpu.VMEM((2, BNS, D, D), jnp.float32),
            pltpu.SemaphoreType.DMA((2,)),
        ],
        compiler_params=pltpu.CompilerParams(
            vmem_limit_bytes=100 * 1024 * 1024),
    )(adj_matrix, obs, maskf, b, W)

    return (mu, eps, fe.reshape(()))
